# baseline (speedup 1.0000x reference)
import functools

import jax
import jax.numpy as jnp
from jax import lax
from jax.experimental import pallas as pl
from jax.experimental.pallas import tpu as pltpu
from jax.experimental.pallas import tpu_sc as plsc

D = 32
NC = 2
NS = 16
NW = NC * NS
SUB = 512
TQ = 4
TB = 8


@jax.jit
def _gather_native(x, weight):
    n_b, n_t = x.shape
    n_sub = (n_b * n_t) // SUB
    per_w = n_sub // NW
    n_pairs = per_w // 2
    mesh = plsc.VectorSubcoreMesh(core_axis_name="c", subcore_axis_name="s")

    @functools.partial(
        pl.kernel,
        mesh=mesh,
        out_type=jax.ShapeDtypeStruct((n_t, 4, 128, 8, 128), jnp.float32),
        scratch_types=[
            pltpu.VMEM((2, 128, TB), jnp.int32),
            pltpu.VMEM((2, TB, 128), jnp.int32),
            pltpu.VMEM((2, TQ, 128, D), jnp.float32),
            pltpu.VMEM((2, TQ, 4, 8, 128), jnp.float32),
            [pltpu.SemaphoreType.DMA] * 2,
            [pltpu.SemaphoreType.DMA] * 2,
            [pltpu.SemaphoreType.DMA] * 2,
        ],
        compiler_params=pltpu.CompilerParams(
            use_tc_tiling_on_sc=False, needs_layout_passes=False
        ),
    )
    def k(x_hbm, table_hbm, out_hbm, idx_v, off_v, rows_v, dst_v,
          sem_i, sem_g, sem_o):
        wid = lax.axis_index("s") * NC + lax.axis_index("c")
        p0 = wid * n_pairs
        iota16 = lax.iota(jnp.int32, 16)

        def idx_start(p, q):
            pb = p0 + p
            btc = pb % 128
            ttr = pb // 128
            pltpu.async_copy(
                x_hbm.at[pl.ds(btc * 128, 128), pl.ds(ttr * TB, TB)],
                idx_v.at[q],
                sem_i[q],
            )

        def idx_wait(q):
            pltpu.make_async_copy(
                x_hbm.at[pl.ds(0, 128), pl.ds(0, TB)], idx_v.at[q], sem_i[q]
            ).wait()

        def idx_transpose(q):
            src = idx_v.at[q]
            for g in range(8):
                base = g * 16
                vs = []
                for t0 in range(TB):
                    tcol = (t0 + iota16) & (TB - 1)
                    vs.append(plsc.load_gather(src, [iota16 + base, tcol]))
                for t0 in range(TB):
                    tcol = (t0 + iota16) & (TB - 1)
                    plsc.store_scatter(
                        off_v.at[q], [tcol, iota16 + base], vs[t0]
                    )

        def gather_start(q, h, b):
            for tq in range(TQ):
                pltpu.async_copy(
                    table_hbm.at[off_v.at[q, h * TQ + tq]],
                    rows_v.at[b, tq],
                    sem_g[b],
                )

        def gather_wait(b):
            for tq in range(TQ):
                pltpu.make_async_copy(
                    table_hbm.at[off_v.at[0, 0]], rows_v.at[b, tq], sem_g[b]
                ).wait()

        def transpose(b):
            for tq in range(TQ):
                rows = rows_v.at[b, tq]

                def dbody(d0, carry):
                    cidx = (d0 + iota16) & (D - 1)
                    dtrv = cidx >> 3
                    ddrv = cidx & 7
                    vs = []
                    for g in range(8):
                        ridx = iota16 + g * 16
                        vs.append(plsc.load_gather(rows, [ridx, cidx]))
                    for g in range(8):
                        plsc.store_scatter(
                            dst_v.at[b, tq],
                            [dtrv, ddrv, iota16 + g * 16],
                            vs[g],
                        )
                    return carry

                lax.fori_loop(0, D, dbody, 0)

        def out_start(n, b):
            pb = p0 + n // 2
            t_base = (pb // 128) * TB + (n % 2) * TQ
            btc = pb % 128
            pltpu.async_copy(
                dst_v.at[b],
                out_hbm.at[pl.ds(t_base, TQ), :, btc],
                sem_o[b],
            )

        def out_wait(b):
            pltpu.make_async_copy(
                dst_v.at[b], out_hbm.at[pl.ds(0, TQ), :, 0], sem_o[b]
            ).wait()

        idx_start(0, 0)
        idx_start(1, 1)
        idx_wait(0)
        idx_transpose(0)
        gather_start(0, 0, 0)

        def body(p, pe):
            gather_wait(0)

            @pl.when(p + 2 < n_pairs)
            def _():
                idx_start(p + 2, pe)

            gather_start(pe, 1, 1)

            @pl.when(p >= 1)
            def _():
                out_wait(0)

            transpose(0)
            out_start(2 * p, 0)

            gather_wait(1)

            @pl.when(p + 1 < n_pairs)
            def _():
                idx_wait(1 - pe)
                idx_transpose(1 - pe)
                gather_start(1 - pe, 0, 0)

            @pl.when(p >= 1)
            def _():
                out_wait(1)

            transpose(1)
            out_start(2 * p + 1, 1)

        def pair2(pp, carry):
            body(2 * pp, 0)
            body(2 * pp + 1, 1)
            return carry

        lax.fori_loop(0, n_pairs // 2, pair2, 0)

        out_wait(0)
        out_wait(1)

    return k(x, weight)


def kernel(x, weight):
    rows, cols = x.shape
    out5 = _gather_native(x.astype(jnp.int32), weight)
    return out5.transpose(2, 4, 0, 1, 3).reshape(rows, cols, D)
